# Initial kernel scaffold; baseline (speedup 1.0000x reference)
import jax
import jax.numpy as jnp
from jax.experimental import pallas as pl


def kernel(genes, X, mask, params):
    raise NotImplementedError("write your pallas kernel here")


# speedup vs baseline: 215.9370x; 215.9370x over previous
import functools

import jax
import jax.numpy as jnp
from jax import lax
from jax.experimental import pallas as pl
from jax.experimental.pallas import tpu as pltpu
from jax.experimental.pallas import tpu_sc as plsc

N = 2048
H = 128
KP = 8
KN = 8
KK = KP + KN
EDF = 64
GVDP = 256
NSM1 = 15
NK = N * KK
RB = 256
NBLK = N // RB
NW = 32
CH = 128
SIGINV = EDF / 8.0
MUSTEP = 8.0 / (EDF - 1)
SQRT_HALF = 0.7071067811865476


def _gelu(x):
    return x * (0.5 * (1.0 + lax.erf(x * SQRT_HALF)))


def _ln(x, g, b):
    m = jnp.mean(x, axis=-1, keepdims=True)
    d = x - m
    v = jnp.mean(d * d, axis=-1, keepdims=True)
    return d * lax.rsqrt(v + 1e-5) * g + b


def _dot(a, b):
    return jnp.dot(a, b, preferred_element_type=jnp.float32)


def _feat_body(xb_ref, xt_ref, g_ref, wg_ref, wv_ref, bv_ref, we_ref, be_ref,
               eidx_ref, hv_ref, he_ref):
    xb = xb_ref[...]
    xt = xt_ref[...]
    x2a = jnp.sum(xt * xt, axis=0, keepdims=True)
    x2b = jnp.sum(xb * xb, axis=1, keepdims=True)
    d2 = x2b + x2a - 2.0 * _dot(xb, xt)
    dd = jnp.sqrt(jnp.maximum(d2, 0.0) + 1e-6)
    iota = lax.broadcasted_iota(jnp.int32, (RB, N), 1)
    mu = MUSTEP * lax.broadcasted_iota(jnp.int32, (1, EDF), 1).astype(jnp.float32)
    we = we_ref[...]
    be = be_ref[...]
    big = jnp.float32(3.0e38)

    def emit(k, val):
        e = jnp.exp(-jnp.square((val - mu) * SIGINV))
        he_ref[:, k, :] = _dot(e, we) + be

    cols = []
    work = dd
    for k in range(KP):
        mn = jnp.min(work, axis=1, keepdims=True)
        idx = jnp.min(jnp.where(work == mn, iota, jnp.int32(N)),
                      axis=1, keepdims=True)
        cols.append(idx)
        emit(k, mn)
        work = jnp.where(iota == idx, big, work)
    work = dd
    for k in range(KN):
        mx = jnp.max(work, axis=1, keepdims=True)
        idx = jnp.min(jnp.where(work == mx, iota, jnp.int32(N)),
                      axis=1, keepdims=True)
        cols.append(idx)
        emit(KP + k, mx)
        work = jnp.where(iota == idx, jnp.float32(-1.0), work)
    eidx_ref[...] = jnp.concatenate(cols, axis=1)

    v = _dot(g_ref[...], wg_ref[...])
    hv_ref[...] = _dot(v, wv_ref[...]) + bv_ref[...]


def _features(xp, xpt, genes_p, wg, wv, bv, we, be):
    full = lambda a: pl.BlockSpec(a.shape, lambda i: (0,) * a.ndim)
    return pl.pallas_call(
        _feat_body,
        grid=(NBLK,),
        in_specs=[
            pl.BlockSpec((RB, 8), lambda i: (i, 0)),
            full(xpt),
            pl.BlockSpec((RB, GVDP), lambda i: (i, 0)),
            full(wg), full(wv), full(bv), full(we), full(be),
        ],
        out_specs=[
            pl.BlockSpec((RB, KK), lambda i: (i, 0)),
            pl.BlockSpec((RB, H), lambda i: (i, 0)),
            pl.BlockSpec((RB, KK, H), lambda i: (i, 0, 0)),
        ],
        out_shape=[
            jax.ShapeDtypeStruct((N, KK), jnp.int32),
            jax.ShapeDtypeStruct((N, H), jnp.float32),
            jax.ShapeDtypeStruct((N, KK, H), jnp.float32),
        ],
    )(xp, xpt, genes_p, wg, wv, bv, we, be)


def _sc_gather(table, idx_flat):
    mesh = plsc.VectorSubcoreMesh(core_axis_name="c", subcore_axis_name="s")
    rows_per_w = NK // NW

    @functools.partial(
        pl.kernel,
        mesh=mesh,
        out_type=jax.ShapeDtypeStruct((NK, H), jnp.float32),
        scratch_types=[
            pltpu.VMEM((CH,), jnp.int32),
            pltpu.VMEM((CH, H), jnp.float32),
            pltpu.SemaphoreType.DMA,
        ],
    )
    def gather_k(table_hbm, idx_hbm, out_hbm, idx_v, rows_v, sem):
        wid = lax.axis_index("s") * 2 + lax.axis_index("c")
        base = wid * rows_per_w
        for j in range(rows_per_w // CH):
            off = base + j * CH
            pltpu.sync_copy(idx_hbm.at[pl.ds(off, CH)], idx_v)
            pltpu.async_copy(table_hbm.at[idx_v], rows_v, sem).wait()
            pltpu.sync_copy(rows_v, out_hbm.at[pl.ds(off, CH)])

    return gather_k(table, idx_flat)


def _message_acc(hv, he_ref, nb_ref, w1v, w1e, w1n, b1, w2, b2, w3, b3,
                 per_edge=None):
    a = _dot(hv, w1v) + b1
    acc = jnp.zeros((RB, H), jnp.float32)
    for k in range(KK):
        t = _dot(he_ref[:, k, :], w1e) + _dot(nb_ref[:, k, :], w1n) + a
        t = _gelu(t)
        t = _gelu(_dot(t, w2) + b2)
        t = _dot(t, w3) + b3
        if per_edge is not None:
            per_edge(k, t)
        else:
            acc = acc + t
    return acc


def _node_body(hv_ref, he_ref, nb_ref, w1v_ref, w1e_ref, w1n_ref, b1_ref,
               w2_ref, b2_ref, w3_ref, b3_ref, g1_ref, gb1_ref,
               mi_ref, mib_ref, mo_ref, mob_ref, g2_ref, gb2_ref,
               out_ref):
    hv = hv_ref[...]
    acc = _message_acc(hv, he_ref, nb_ref, w1v_ref[...], w1e_ref[...],
                       w1n_ref[...], b1_ref[...], w2_ref[...], b2_ref[...],
                       w3_ref[...], b3_ref[...])
    hv = _ln(hv + acc * (1.0 / KK), g1_ref[...], gb1_ref[...])
    ffn = _dot(_gelu(_dot(hv, mi_ref[...]) + mib_ref[...]), mo_ref[...]) \
        + mob_ref[...]
    out_ref[...] = _ln(hv + ffn, g2_ref[...], gb2_ref[...])


def _dec_body(hv_ref, he_ref, nb_ref, w1v_ref, w1e_ref, w1n_ref, b1_ref,
              w2_ref, b2_ref, w3_ref, b3_ref, g1_ref, gb1_ref,
              mi_ref, mib_ref, mo_ref, mob_ref, g2_ref, gb2_ref,
              wr_ref, br_ref, out_ref):
    hv = hv_ref[...]
    acc = _message_acc(hv, he_ref, nb_ref, w1v_ref[...], w1e_ref[...],
                       w1n_ref[...], b1_ref[...], w2_ref[...], b2_ref[...],
                       w3_ref[...], b3_ref[...])
    hv = _ln(hv + acc * (1.0 / KK), g1_ref[...], gb1_ref[...])
    ffn = _dot(_gelu(_dot(hv, mi_ref[...]) + mib_ref[...]), mo_ref[...]) \
        + mob_ref[...]
    hv = _ln(hv + ffn, g2_ref[...], gb2_ref[...])
    out_ref[...] = _dot(hv, wr_ref[...]) + br_ref[...]


def _edge_body(hv_ref, he_ref, nb_ref, w1v_ref, w1e_ref, w1n_ref, b1_ref,
               w2_ref, b2_ref, w3_ref, b3_ref, g3_ref, gb3_ref, out_ref):
    hv = hv_ref[...]
    g3 = g3_ref[...]
    gb3 = gb3_ref[...]

    def upd(k, t):
        out_ref[:, k, :] = _ln(he_ref[:, k, :] + t, g3, gb3)

    _message_acc(hv, he_ref, nb_ref, w1v_ref[...], w1e_ref[...], w1n_ref[...],
                 b1_ref[...], w2_ref[...], b2_ref[...], w3_ref[...],
                 b3_ref[...], per_edge=upd)


def _mp_call(body, hv, he, nb, weights, out_shape, out_spec):
    full = lambda a: pl.BlockSpec(a.shape, lambda i: (0,) * a.ndim)
    return pl.pallas_call(
        body,
        grid=(NBLK,),
        in_specs=[
            pl.BlockSpec((RB, H), lambda i: (i, 0)),
            pl.BlockSpec((RB, KK, H), lambda i: (i, 0, 0)),
            pl.BlockSpec((RB, KK, H), lambda i: (i, 0, 0)),
        ] + [full(w) for w in weights],
        out_specs=out_spec,
        out_shape=out_shape,
    )(hv, he, nb, *weights)


def _readout_body(hv_ref, he_ref, nb_ref, w1v_ref, w1e_ref, w1n_ref, b1_ref,
                  w2_ref, b2_ref, w3_ref, b3_ref, out_ref, h_scr):
    hv = hv_ref[...]
    w1v = w1v_ref[...]
    w1e = w1e_ref[...]
    w1n = w1n_ref[...]
    av = jnp.sum(hv * w1v, axis=1, keepdims=True) + b1_ref[...]
    for k in range(KK):
        s = av + jnp.sum(he_ref[:, k, :] * w1e, axis=1, keepdims=True) \
            + jnp.sum(nb_ref[:, k, :] * w1n, axis=1, keepdims=True)
        h_scr[:, k:k + 1] = _gelu(s)
    t = jnp.sum(h_scr[...] * w2_ref[...], axis=1, keepdims=True) + b2_ref[...]
    t = _gelu(t)
    part = jnp.sum(t * w3_ref[...], axis=0, keepdims=True)
    i = pl.program_id(0)

    @pl.when(i == 0)
    def _():
        out_ref[...] = part + b3_ref[...]

    @pl.when(i != 0)
    def _():
        out_ref[...] += part


def _readout(hv, he, nb, w1v, w1e, w1n, b1, w2, b2, w3, b3):
    full = lambda a: pl.BlockSpec(a.shape, lambda i: (0,) * a.ndim)
    return pl.pallas_call(
        _readout_body,
        grid=(NBLK,),
        in_specs=[
            pl.BlockSpec((RB, H), lambda i: (i, 0)),
            pl.BlockSpec((RB, KK, H), lambda i: (i, 0, 0)),
            pl.BlockSpec((RB, KK, H), lambda i: (i, 0, 0)),
            full(w1v), full(w1e), full(w1n), full(b1),
            full(w2), full(b2),
            pl.BlockSpec((RB, NSM1), lambda i: (i, 0)),
            full(b3),
        ],
        out_specs=pl.BlockSpec((1, NSM1), lambda i: (0, 0)),
        out_shape=jax.ShapeDtypeStruct((1, NSM1), jnp.float32),
        scratch_shapes=[pltpu.VMEM((RB, KK), jnp.float32)],
    )(hv, he, nb, w1v, w1e, w1n, b1, w2, b2, w3, b3)


def _row(v):
    return v.reshape(1, -1)


def _mlp_weights(p, names):
    w1, w2, w3 = (p[n] for n in names)
    return [w1["W"][0:H], w1["W"][H:2 * H], w1["W"][2 * H:3 * H],
            _row(w1["b"]), w2["W"], _row(w2["b"]), w3["W"], _row(w3["b"])]


def kernel(genes, X, mask, params):
    p = params
    g2 = jnp.pad(genes[0], ((0, 0), (0, GVDP - genes.shape[-1])))
    xp = jnp.pad(X[0], ((0, 0), (0, 8 - X.shape[-1])))
    xpt = xp.T
    wg = jnp.pad(p["W_genes"], ((0, GVDP - p["W_genes"].shape[0]), (0, 0)))

    eidx, hv, he = _features(
        xp, xpt, g2, wg, p["W_v"]["W"], _row(p["W_v"]["b"]),
        p["W_e"]["W"], _row(p["W_e"]["b"]))
    idx_flat = eidx.reshape(NK)

    node_sp = pl.BlockSpec((RB, H), lambda i: (i, 0))
    edge_sp = pl.BlockSpec((RB, KK, H), lambda i: (i, 0, 0))
    node_sh = jax.ShapeDtypeStruct((N, H), jnp.float32)
    edge_sh = jax.ShapeDtypeStruct((N, KK, H), jnp.float32)

    for ep in p["enc"]:
        nw = _mlp_weights(ep, ("W1", "W2", "W3")) + [
            _row(ep["ln1_g"]), _row(ep["ln1_b"]),
            ep["mix_in"]["W"], _row(ep["mix_in"]["b"]),
            ep["mix_out"]["W"], _row(ep["mix_out"]["b"]),
            _row(ep["ln2_g"]), _row(ep["ln2_b"])]
        nb = _sc_gather(hv, idx_flat).reshape(N, KK, H)
        hv = _mp_call(_node_body, hv, he, nb, nw, node_sh, node_sp)
        ew = _mlp_weights(ep, ("W11", "W12", "W13")) + [
            _row(ep["ln3_g"]), _row(ep["ln3_b"])]
        nb = _sc_gather(hv, idx_flat).reshape(N, KK, H)
        he = _mp_call(_edge_body, hv, he, nb, ew, edge_sh, edge_sp)

    nb = _sc_gather(hv, idx_flat).reshape(N, KK, H)

    c1 = p["W_cell1"]
    logits = _readout(
        hv, he, nb,
        _row(c1["W"][0:H, 0]), _row(c1["W"][H:2 * H, 0]),
        _row(c1["W"][2 * H:3 * H, 0]), _row(c1["b"]),
        _row(p["W_cell2"]["W"][:, 0]), _row(p["W_cell2"]["b"]),
        p["W_cell3"]["W"], _row(p["W_cell3"]["b"]))

    dp = p["dec"][0]
    dw = _mlp_weights(dp, ("W1", "W2", "W3")) + [
        _row(dp["ln1_g"]), _row(dp["ln1_b"]),
        dp["mix_in"]["W"], _row(dp["mix_in"]["b"]),
        dp["mix_out"]["W"], _row(dp["mix_out"]["b"]),
        _row(dp["ln2_g"]), _row(dp["ln2_b"]),
        p["W_recover"]["W"], _row(p["W_recover"]["b"])]
    recover = _mp_call(
        _dec_body, hv, he, nb, dw,
        jax.ShapeDtypeStruct((N, p["W_recover"]["W"].shape[1]), jnp.float32),
        pl.BlockSpec((RB, p["W_recover"]["W"].shape[1]), lambda i: (i, 0)))

    return recover[None], logits


# speedup vs baseline: 216.6073x; 1.0031x over previous
import functools

import jax
import jax.numpy as jnp
from jax import lax
from jax.experimental import pallas as pl
from jax.experimental.pallas import tpu as pltpu
from jax.experimental.pallas import tpu_sc as plsc

N = 2048
H = 128
KP = 8
KN = 8
KK = KP + KN
EDF = 64
GVDP = 256
NSM1 = 15
NK = N * KK
RB = 256
NBLK = N // RB
NW = 32
CH = 128
SIGINV = EDF / 8.0
MUSTEP = 8.0 / (EDF - 1)
SQRT_HALF = 0.7071067811865476


def _gelu(x):
    return x * (0.5 * (1.0 + lax.erf(x * SQRT_HALF)))


def _ln(x, g, b):
    m = jnp.mean(x, axis=-1, keepdims=True)
    d = x - m
    v = jnp.mean(d * d, axis=-1, keepdims=True)
    return d * lax.rsqrt(v + 1e-5) * g + b


def _dot(a, b):
    return jnp.dot(a, b, preferred_element_type=jnp.float32)


def _feat_body(xb_ref, xt_ref, g_ref, wg_ref, wv_ref, bv_ref, we_ref, be_ref,
               eidx_ref, hv_ref, he_ref):
    xb = xb_ref[...]
    xt = xt_ref[...]
    x2a = jnp.sum(xt * xt, axis=0, keepdims=True)
    x2b = jnp.sum(xb * xb, axis=1, keepdims=True)
    d2 = x2b + x2a - 2.0 * _dot(xb, xt)
    dd = jnp.sqrt(jnp.maximum(d2, 0.0) + 1e-6)
    iota = lax.broadcasted_iota(jnp.int32, (RB, N), 1)
    mu = MUSTEP * lax.broadcasted_iota(jnp.int32, (1, EDF), 1).astype(jnp.float32)
    we = we_ref[...]
    be = be_ref[...]
    big = jnp.float32(3.0e38)

    def emit(k, val):
        e = jnp.exp(-jnp.square((val - mu) * SIGINV))
        he_ref[:, k, :] = _dot(e, we) + be

    cols = []
    work = dd
    for k in range(KP):
        mn = jnp.min(work, axis=1, keepdims=True)
        idx = jnp.min(jnp.where(work == mn, iota, jnp.int32(N)),
                      axis=1, keepdims=True)
        cols.append(idx)
        emit(k, mn)
        work = jnp.where(iota == idx, big, work)
    work = dd
    for k in range(KN):
        mx = jnp.max(work, axis=1, keepdims=True)
        idx = jnp.min(jnp.where(work == mx, iota, jnp.int32(N)),
                      axis=1, keepdims=True)
        cols.append(idx)
        emit(KP + k, mx)
        work = jnp.where(iota == idx, jnp.float32(-1.0), work)
    eidx_ref[...] = jnp.concatenate(cols, axis=1)

    v = _dot(g_ref[...], wg_ref[...])
    hv_ref[...] = _dot(v, wv_ref[...]) + bv_ref[...]


def _features(xp, xpt, genes_p, wg, wv, bv, we, be):
    full = lambda a: pl.BlockSpec(a.shape, lambda i: (0,) * a.ndim)
    return pl.pallas_call(
        _feat_body,
        grid=(NBLK,),
        in_specs=[
            pl.BlockSpec((RB, 8), lambda i: (i, 0)),
            full(xpt),
            pl.BlockSpec((RB, GVDP), lambda i: (i, 0)),
            full(wg), full(wv), full(bv), full(we), full(be),
        ],
        out_specs=[
            pl.BlockSpec((RB, KK), lambda i: (i, 0)),
            pl.BlockSpec((RB, H), lambda i: (i, 0)),
            pl.BlockSpec((RB, KK, H), lambda i: (i, 0, 0)),
        ],
        out_shape=[
            jax.ShapeDtypeStruct((N, KK), jnp.int32),
            jax.ShapeDtypeStruct((N, H), jnp.float32),
            jax.ShapeDtypeStruct((N, KK, H), jnp.float32),
        ],
    )(xp, xpt, genes_p, wg, wv, bv, we, be)


def _sc_gather(table, idx2d):
    mesh = plsc.VectorSubcoreMesh(core_axis_name="c", subcore_axis_name="s")
    rows_per_w = NK // NW
    nch = rows_per_w // CH
    nb = 4

    @functools.partial(
        pl.kernel,
        mesh=mesh,
        out_type=jax.ShapeDtypeStruct((NK, H), jnp.float32),
        scratch_types=[
            pltpu.VMEM((nch, CH), jnp.int32),
        ] + [pltpu.VMEM((CH, H), jnp.float32) for _ in range(nb)] + [
            pltpu.SemaphoreType.DMA,
            pltpu.SemaphoreType.DMA,
        ],
    )
    def gather_k(table_hbm, idx_hbm, out_hbm, idx_v, *rest):
        bufs = rest[:nb]
        gsem, wsem = rest[nb], rest[nb + 1]
        wid = lax.axis_index("s") * 2 + lax.axis_index("c")
        base = wid * rows_per_w
        crow = wid * nch
        pltpu.sync_copy(idx_hbm.at[pl.ds(crow, nch)], idx_v)
        gh = [None] * nch
        wh = [None] * nch
        for j in range(nch):
            if j >= nb:
                wh[j - nb].wait()
            gh[j] = pltpu.async_copy(table_hbm.at[idx_v.at[j]], bufs[j % nb],
                                     gsem)
            if j >= 1:
                gh[j - 1].wait()
                wh[j - 1] = pltpu.async_copy(
                    bufs[(j - 1) % nb],
                    out_hbm.at[pl.ds(base + (j - 1) * CH, CH)], wsem)
        gh[nch - 1].wait()
        wh[nch - 1] = pltpu.async_copy(
            bufs[(nch - 1) % nb],
            out_hbm.at[pl.ds(base + (nch - 1) * CH, CH)], wsem)
        for j in range(max(0, nch - nb), nch):
            if j < nch - 1:
                wh[j].wait()
        wh[nch - 1].wait()

    return gather_k(table, idx2d)


def _message_acc(hv, he_ref, nb_ref, w1v, w1e, w1n, b1, w2, b2, w3, b3,
                 per_edge=None):
    a = _dot(hv, w1v) + b1
    acc = jnp.zeros((RB, H), jnp.float32)
    for k in range(KK):
        t = _dot(he_ref[:, k, :], w1e) + _dot(nb_ref[:, k, :], w1n) + a
        t = _gelu(t)
        t = _gelu(_dot(t, w2) + b2)
        t = _dot(t, w3) + b3
        if per_edge is not None:
            per_edge(k, t)
        else:
            acc = acc + t
    return acc


def _node_body(hv_ref, he_ref, nb_ref, w1v_ref, w1e_ref, w1n_ref, b1_ref,
               w2_ref, b2_ref, w3_ref, b3_ref, g1_ref, gb1_ref,
               mi_ref, mib_ref, mo_ref, mob_ref, g2_ref, gb2_ref,
               out_ref):
    hv = hv_ref[...]
    acc = _message_acc(hv, he_ref, nb_ref, w1v_ref[...], w1e_ref[...],
                       w1n_ref[...], b1_ref[...], w2_ref[...], b2_ref[...],
                       w3_ref[...], b3_ref[...])
    hv = _ln(hv + acc * (1.0 / KK), g1_ref[...], gb1_ref[...])
    ffn = _dot(_gelu(_dot(hv, mi_ref[...]) + mib_ref[...]), mo_ref[...]) \
        + mob_ref[...]
    out_ref[...] = _ln(hv + ffn, g2_ref[...], gb2_ref[...])


def _dec_body(hv_ref, he_ref, nb_ref, w1v_ref, w1e_ref, w1n_ref, b1_ref,
              w2_ref, b2_ref, w3_ref, b3_ref, g1_ref, gb1_ref,
              mi_ref, mib_ref, mo_ref, mob_ref, g2_ref, gb2_ref,
              wr_ref, br_ref, out_ref):
    hv = hv_ref[...]
    acc = _message_acc(hv, he_ref, nb_ref, w1v_ref[...], w1e_ref[...],
                       w1n_ref[...], b1_ref[...], w2_ref[...], b2_ref[...],
                       w3_ref[...], b3_ref[...])
    hv = _ln(hv + acc * (1.0 / KK), g1_ref[...], gb1_ref[...])
    ffn = _dot(_gelu(_dot(hv, mi_ref[...]) + mib_ref[...]), mo_ref[...]) \
        + mob_ref[...]
    hv = _ln(hv + ffn, g2_ref[...], gb2_ref[...])
    out_ref[...] = _dot(hv, wr_ref[...]) + br_ref[...]


def _edge_body(hv_ref, he_ref, nb_ref, w1v_ref, w1e_ref, w1n_ref, b1_ref,
               w2_ref, b2_ref, w3_ref, b3_ref, g3_ref, gb3_ref, out_ref):
    hv = hv_ref[...]
    g3 = g3_ref[...]
    gb3 = gb3_ref[...]

    def upd(k, t):
        out_ref[:, k, :] = _ln(he_ref[:, k, :] + t, g3, gb3)

    _message_acc(hv, he_ref, nb_ref, w1v_ref[...], w1e_ref[...], w1n_ref[...],
                 b1_ref[...], w2_ref[...], b2_ref[...], w3_ref[...],
                 b3_ref[...], per_edge=upd)


def _mp_call(body, hv, he, nb, weights, out_shape, out_spec):
    full = lambda a: pl.BlockSpec(a.shape, lambda i: (0,) * a.ndim)
    return pl.pallas_call(
        body,
        grid=(NBLK,),
        in_specs=[
            pl.BlockSpec((RB, H), lambda i: (i, 0)),
            pl.BlockSpec((RB, KK, H), lambda i: (i, 0, 0)),
            pl.BlockSpec((RB, KK, H), lambda i: (i, 0, 0)),
        ] + [full(w) for w in weights],
        out_specs=out_spec,
        out_shape=out_shape,
    )(hv, he, nb, *weights)


def _readout_body(hv_ref, he_ref, nb_ref, w1v_ref, w1e_ref, w1n_ref, b1_ref,
                  w2_ref, b2_ref, w3_ref, b3_ref, out_ref, h_scr):
    hv = hv_ref[...]
    w1v = w1v_ref[...]
    w1e = w1e_ref[...]
    w1n = w1n_ref[...]
    av = jnp.sum(hv * w1v, axis=1, keepdims=True) + b1_ref[...]
    for k in range(KK):
        s = av + jnp.sum(he_ref[:, k, :] * w1e, axis=1, keepdims=True) \
            + jnp.sum(nb_ref[:, k, :] * w1n, axis=1, keepdims=True)
        h_scr[:, k:k + 1] = _gelu(s)
    t = jnp.sum(h_scr[...] * w2_ref[...], axis=1, keepdims=True) + b2_ref[...]
    t = _gelu(t)
    part = jnp.sum(t * w3_ref[...], axis=0, keepdims=True)
    i = pl.program_id(0)

    @pl.when(i == 0)
    def _():
        out_ref[...] = part + b3_ref[...]

    @pl.when(i != 0)
    def _():
        out_ref[...] += part


def _readout(hv, he, nb, w1v, w1e, w1n, b1, w2, b2, w3, b3):
    full = lambda a: pl.BlockSpec(a.shape, lambda i: (0,) * a.ndim)
    return pl.pallas_call(
        _readout_body,
        grid=(NBLK,),
        in_specs=[
            pl.BlockSpec((RB, H), lambda i: (i, 0)),
            pl.BlockSpec((RB, KK, H), lambda i: (i, 0, 0)),
            pl.BlockSpec((RB, KK, H), lambda i: (i, 0, 0)),
            full(w1v), full(w1e), full(w1n), full(b1),
            full(w2), full(b2),
            pl.BlockSpec((RB, NSM1), lambda i: (i, 0)),
            full(b3),
        ],
        out_specs=pl.BlockSpec((1, NSM1), lambda i: (0, 0)),
        out_shape=jax.ShapeDtypeStruct((1, NSM1), jnp.float32),
        scratch_shapes=[pltpu.VMEM((RB, KK), jnp.float32)],
    )(hv, he, nb, w1v, w1e, w1n, b1, w2, b2, w3, b3)


def _row(v):
    return v.reshape(1, -1)


def _mlp_weights(p, names):
    w1, w2, w3 = (p[n] for n in names)
    return [w1["W"][0:H], w1["W"][H:2 * H], w1["W"][2 * H:3 * H],
            _row(w1["b"]), w2["W"], _row(w2["b"]), w3["W"], _row(w3["b"])]


def kernel(genes, X, mask, params):
    p = params
    g2 = jnp.pad(genes[0], ((0, 0), (0, GVDP - genes.shape[-1])))
    xp = jnp.pad(X[0], ((0, 0), (0, 8 - X.shape[-1])))
    xpt = xp.T
    wg = jnp.pad(p["W_genes"], ((0, GVDP - p["W_genes"].shape[0]), (0, 0)))

    eidx, hv, he = _features(
        xp, xpt, g2, wg, p["W_v"]["W"], _row(p["W_v"]["b"]),
        p["W_e"]["W"], _row(p["W_e"]["b"]))
    idx_flat = eidx.reshape(NK // CH, CH)

    node_sp = pl.BlockSpec((RB, H), lambda i: (i, 0))
    edge_sp = pl.BlockSpec((RB, KK, H), lambda i: (i, 0, 0))
    node_sh = jax.ShapeDtypeStruct((N, H), jnp.float32)
    edge_sh = jax.ShapeDtypeStruct((N, KK, H), jnp.float32)

    for ep in p["enc"]:
        nw = _mlp_weights(ep, ("W1", "W2", "W3")) + [
            _row(ep["ln1_g"]), _row(ep["ln1_b"]),
            ep["mix_in"]["W"], _row(ep["mix_in"]["b"]),
            ep["mix_out"]["W"], _row(ep["mix_out"]["b"]),
            _row(ep["ln2_g"]), _row(ep["ln2_b"])]
        nb = _sc_gather(hv, idx_flat).reshape(N, KK, H)
        hv = _mp_call(_node_body, hv, he, nb, nw, node_sh, node_sp)
        ew = _mlp_weights(ep, ("W11", "W12", "W13")) + [
            _row(ep["ln3_g"]), _row(ep["ln3_b"])]
        nb = _sc_gather(hv, idx_flat).reshape(N, KK, H)
        he = _mp_call(_edge_body, hv, he, nb, ew, edge_sh, edge_sp)

    nb = _sc_gather(hv, idx_flat).reshape(N, KK, H)

    c1 = p["W_cell1"]
    logits = _readout(
        hv, he, nb,
        _row(c1["W"][0:H, 0]), _row(c1["W"][H:2 * H, 0]),
        _row(c1["W"][2 * H:3 * H, 0]), _row(c1["b"]),
        _row(p["W_cell2"]["W"][:, 0]), _row(p["W_cell2"]["b"]),
        p["W_cell3"]["W"], _row(p["W_cell3"]["b"]))

    dp = p["dec"][0]
    dw = _mlp_weights(dp, ("W1", "W2", "W3")) + [
        _row(dp["ln1_g"]), _row(dp["ln1_b"]),
        dp["mix_in"]["W"], _row(dp["mix_in"]["b"]),
        dp["mix_out"]["W"], _row(dp["mix_out"]["b"]),
        _row(dp["ln2_g"]), _row(dp["ln2_b"]),
        p["W_recover"]["W"], _row(p["W_recover"]["b"])]
    recover = _mp_call(
        _dec_body, hv, he, nb, dw,
        jax.ShapeDtypeStruct((N, p["W_recover"]["W"].shape[1]), jnp.float32),
        pl.BlockSpec((RB, p["W_recover"]["W"].shape[1]), lambda i: (i, 0)))

    return recover[None], logits


# speedup vs baseline: 300.7769x; 1.3886x over previous
import functools

import jax
import jax.numpy as jnp
from jax import lax
from jax.experimental import pallas as pl
from jax.experimental.pallas import tpu as pltpu
from jax.experimental.pallas import tpu_sc as plsc

N = 2048
H = 128
KP = 8
KN = 8
KK = KP + KN
EDF = 64
GVDP = 256
NSM1 = 15
NK = N * KK
RB = 256
ER = KK * RB
NBLK = N // RB
NW = 32
CH = 128
SIGINV = EDF / 8.0
MUSTEP = 8.0 / (EDF - 1)
SQRT_HALF = 0.7071067811865476


def _gelu(x):
    return x * (0.5 * (1.0 + lax.erf(x * SQRT_HALF)))


def _ln(x, g, b):
    m = jnp.mean(x, axis=-1, keepdims=True)
    d = x - m
    v = jnp.mean(d * d, axis=-1, keepdims=True)
    return d * lax.rsqrt(v + 1e-5) * g + b


def _dot(a, b):
    return jnp.dot(a, b, preferred_element_type=jnp.float32)


def _feat_body(xb_ref, xt_ref, g_ref, wg_ref, wv_ref, bv_ref, we_ref, be_ref,
               eidx_ref, hv_ref, he_ref):
    xb = xb_ref[...]
    xt = xt_ref[...]
    x2a = jnp.sum(xt * xt, axis=0, keepdims=True)
    x2b = jnp.sum(xb * xb, axis=1, keepdims=True)
    d2 = x2b + x2a - 2.0 * _dot(xb, xt)
    dd = jnp.sqrt(jnp.maximum(d2, 0.0) + 1e-6)
    iota = lax.broadcasted_iota(jnp.int32, (RB, N), 1)
    mu = MUSTEP * lax.broadcasted_iota(jnp.int32, (1, EDF), 1).astype(jnp.float32)
    we = we_ref[...]
    be = be_ref[...]
    big = jnp.float32(3.0e38)

    def emit(k, val):
        e = jnp.exp(-jnp.square((val - mu) * SIGINV))
        he_ref[k, :, :] = _dot(e, we) + be

    cols = []
    work = dd
    for k in range(KP):
        mn = jnp.min(work, axis=1, keepdims=True)
        idx = jnp.min(jnp.where(work == mn, iota, jnp.int32(N)),
                      axis=1, keepdims=True)
        cols.append(idx)
        emit(k, mn)
        work = jnp.where(iota == idx, big, work)
    work = dd
    for k in range(KN):
        mx = jnp.max(work, axis=1, keepdims=True)
        idx = jnp.min(jnp.where(work == mx, iota, jnp.int32(N)),
                      axis=1, keepdims=True)
        cols.append(idx)
        emit(KP + k, mx)
        work = jnp.where(iota == idx, jnp.float32(-1.0), work)
    eidx_ref[...] = jnp.concatenate(cols, axis=1)

    v = _dot(g_ref[...], wg_ref[...])
    hv_ref[...] = _dot(v, wv_ref[...]) + bv_ref[...]


def _features(xp, xpt, genes_p, wg, wv, bv, we, be):
    full = lambda a: pl.BlockSpec(a.shape, lambda i: (0,) * a.ndim)
    return pl.pallas_call(
        _feat_body,
        grid=(NBLK,),
        in_specs=[
            pl.BlockSpec((RB, 8), lambda i: (i, 0)),
            full(xpt),
            pl.BlockSpec((RB, GVDP), lambda i: (i, 0)),
            full(wg), full(wv), full(bv), full(we), full(be),
        ],
        out_specs=[
            pl.BlockSpec((RB, KK), lambda i: (i, 0)),
            pl.BlockSpec((RB, H), lambda i: (i, 0)),
            pl.BlockSpec((KK, RB, H), lambda i: (0, i, 0)),
        ],
        out_shape=[
            jax.ShapeDtypeStruct((N, KK), jnp.int32),
            jax.ShapeDtypeStruct((N, H), jnp.float32),
            jax.ShapeDtypeStruct((KK, N, H), jnp.float32),
        ],
    )(xp, xpt, genes_p, wg, wv, bv, we, be)


def _sc_gather(table, idx2d):
    mesh = plsc.VectorSubcoreMesh(core_axis_name="c", subcore_axis_name="s")
    rows_per_w = NK // NW
    nch = rows_per_w // CH
    nb = 4

    @functools.partial(
        pl.kernel,
        mesh=mesh,
        out_type=jax.ShapeDtypeStruct((NK, H), jnp.float32),
        scratch_types=[
            pltpu.VMEM((nch, CH), jnp.int32),
        ] + [pltpu.VMEM((CH, H), jnp.float32) for _ in range(nb)] + [
            pltpu.SemaphoreType.DMA,
            pltpu.SemaphoreType.DMA,
        ],
    )
    def gather_k(table_hbm, idx_hbm, out_hbm, idx_v, *rest):
        bufs = rest[:nb]
        gsem, wsem = rest[nb], rest[nb + 1]
        wid = lax.axis_index("s") * 2 + lax.axis_index("c")
        base = wid * rows_per_w
        crow = wid * nch
        pltpu.sync_copy(idx_hbm.at[pl.ds(crow, nch)], idx_v)
        gh = [None] * nch
        wh = [None] * nch
        for j in range(nch):
            if j >= nb:
                wh[j - nb].wait()
            gh[j] = pltpu.async_copy(table_hbm.at[idx_v.at[j]], bufs[j % nb],
                                     gsem)
            if j >= 1:
                gh[j - 1].wait()
                wh[j - 1] = pltpu.async_copy(
                    bufs[(j - 1) % nb],
                    out_hbm.at[pl.ds(base + (j - 1) * CH, CH)], wsem)
        gh[nch - 1].wait()
        wh[nch - 1] = pltpu.async_copy(
            bufs[(nch - 1) % nb],
            out_hbm.at[pl.ds(base + (nch - 1) * CH, CH)], wsem)
        for j in range(max(0, nch - nb), nch):
            wh[j].wait()

    return gather_k(table, idx2d)


def _message_flat(hv, he, nbr, w1v, w1e, w1n, b1, w2, b2, w3, b3):
    a = _dot(hv, w1v) + b1
    a_rep = jnp.broadcast_to(a[None], (KK, RB, H)).reshape(ER, H)
    t = _dot(he, w1e) + _dot(nbr, w1n) + a_rep
    t = _gelu(t)
    t = _gelu(_dot(t, w2) + b2)
    return _dot(t, w3) + b3


def _node_common(hv, he_ref, nb_ref, w1v_ref, w1e_ref, w1n_ref, b1_ref,
                 w2_ref, b2_ref, w3_ref, b3_ref, g1_ref, gb1_ref,
                 mi_ref, mib_ref, mo_ref, mob_ref, g2_ref, gb2_ref):
    he = he_ref[...].reshape(ER, H)
    nbr = nb_ref[...].reshape(ER, H)
    t = _message_flat(hv, he, nbr, w1v_ref[...], w1e_ref[...], w1n_ref[...],
                      b1_ref[...], w2_ref[...], b2_ref[...], w3_ref[...],
                      b3_ref[...])
    acc = jnp.sum(t.reshape(KK, RB, H), axis=0)
    hv = _ln(hv + acc * (1.0 / KK), g1_ref[...], gb1_ref[...])
    ffn = _dot(_gelu(_dot(hv, mi_ref[...]) + mib_ref[...]), mo_ref[...]) \
        + mob_ref[...]
    return _ln(hv + ffn, g2_ref[...], gb2_ref[...])


def _node_body(hv_ref, he_ref, nb_ref, *refs):
    out_ref = refs[-1]
    out_ref[...] = _node_common(hv_ref[...], he_ref, nb_ref, *refs[:-1])


def _edge_body(hv_ref, he_ref, nb_ref, w1v_ref, w1e_ref, w1n_ref, b1_ref,
               w2_ref, b2_ref, w3_ref, b3_ref, g3_ref, gb3_ref, out_ref):
    he = he_ref[...].reshape(ER, H)
    nbr = nb_ref[...].reshape(ER, H)
    t = _message_flat(hv_ref[...], he, nbr, w1v_ref[...], w1e_ref[...],
                      w1n_ref[...], b1_ref[...], w2_ref[...], b2_ref[...],
                      w3_ref[...], b3_ref[...])
    out_ref[...] = _ln(he + t, g3_ref[...], gb3_ref[...]).reshape(KK, RB, H)


def _dec_body(hv_ref, he_ref, nb_ref, w1v_ref, w1e_ref, w1n_ref, b1_ref,
              w2_ref, b2_ref, w3_ref, b3_ref, g1_ref, gb1_ref,
              mi_ref, mib_ref, mo_ref, mob_ref, g2_ref, gb2_ref,
              wr_ref, br_ref,
              rv1_ref, re1_ref, rn1_ref, rb1_ref, w2rep_ref, rb2_ref,
              w3c_ref, rb3_ref,
              rec_ref, log_ref):
    hv = hv_ref[...]
    he = he_ref[...].reshape(ER, H)
    nbr = nb_ref[...].reshape(ER, H)

    av = jnp.sum(hv * rv1_ref[...], axis=1, keepdims=True)
    s = jnp.sum(he * re1_ref[...], axis=1, keepdims=True) \
        + jnp.sum(nbr * rn1_ref[...], axis=1, keepdims=True)
    av_rep = jnp.broadcast_to(av[None], (KK, RB, 1)).reshape(ER, 1)
    h = _gelu(s + av_rep + rb1_ref[...])
    tsum = jnp.sum((h * w2rep_ref[...]).reshape(KK, RB, 1), axis=0)
    t2 = _gelu(tsum + rb2_ref[...])
    i = pl.program_id(0)
    w3c = w3c_ref[pl.ds(i * RB, RB), :]
    part = jnp.sum(t2 * w3c, axis=0, keepdims=True)

    @pl.when(i == 0)
    def _():
        log_ref[...] = part + rb3_ref[...]

    @pl.when(i != 0)
    def _():
        log_ref[...] += part

    hv2 = _node_common(hv, he_ref, nb_ref, w1v_ref, w1e_ref, w1n_ref, b1_ref,
                       w2_ref, b2_ref, w3_ref, b3_ref, g1_ref, gb1_ref,
                       mi_ref, mib_ref, mo_ref, mob_ref, g2_ref, gb2_ref)
    rec_ref[...] = _dot(hv2, wr_ref[...]) + br_ref[...]


def _mp_call(body, hv, he, nb3, weights, out_shape, out_spec,
             scratch_shapes=()):
    full = lambda a: pl.BlockSpec(a.shape, lambda i: (0,) * a.ndim)
    return pl.pallas_call(
        body,
        grid=(NBLK,),
        in_specs=[
            pl.BlockSpec((RB, H), lambda i: (i, 0)),
            pl.BlockSpec((KK, RB, H), lambda i: (0, i, 0)),
            pl.BlockSpec((KK, RB, H), lambda i: (0, i, 0)),
        ] + [full(w) for w in weights],
        out_specs=out_spec,
        out_shape=out_shape,
        scratch_shapes=list(scratch_shapes),
    )(hv, he, nb3, *weights)


def _row(v):
    return v.reshape(1, -1)


def _mlp_weights(p, names):
    w1, w2, w3 = (p[n] for n in names)
    return [w1["W"][0:H], w1["W"][H:2 * H], w1["W"][2 * H:3 * H],
            _row(w1["b"]), w2["W"], _row(w2["b"]), w3["W"], _row(w3["b"])]


def kernel(genes, X, mask, params):
    p = params
    g2 = jnp.pad(genes[0], ((0, 0), (0, GVDP - genes.shape[-1])))
    xp = jnp.pad(X[0], ((0, 0), (0, 8 - X.shape[-1])))
    xpt = xp.T
    wg = jnp.pad(p["W_genes"], ((0, GVDP - p["W_genes"].shape[0]), (0, 0)))

    eidx, hv, he = _features(
        xp, xpt, g2, wg, p["W_v"]["W"], _row(p["W_v"]["b"]),
        p["W_e"]["W"], _row(p["W_e"]["b"]))
    idx2d = eidx.T.reshape(NK // CH, CH)

    node_sp = pl.BlockSpec((RB, H), lambda i: (i, 0))
    edge_sp = pl.BlockSpec((KK, RB, H), lambda i: (0, i, 0))
    node_sh = jax.ShapeDtypeStruct((N, H), jnp.float32)
    edge_sh = jax.ShapeDtypeStruct((KK, N, H), jnp.float32)

    nb3 = _sc_gather(hv, idx2d).reshape(KK, N, H)
    for ep in p["enc"]:
        nw = _mlp_weights(ep, ("W1", "W2", "W3")) + [
            _row(ep["ln1_g"]), _row(ep["ln1_b"]),
            ep["mix_in"]["W"], _row(ep["mix_in"]["b"]),
            ep["mix_out"]["W"], _row(ep["mix_out"]["b"]),
            _row(ep["ln2_g"]), _row(ep["ln2_b"])]
        hv = _mp_call(_node_body, hv, he, nb3, nw, node_sh, node_sp)
        ew = _mlp_weights(ep, ("W11", "W12", "W13")) + [
            _row(ep["ln3_g"]), _row(ep["ln3_b"])]
        nb3 = _sc_gather(hv, idx2d).reshape(KK, N, H)
        he = _mp_call(_edge_body, hv, he, nb3, ew, edge_sh, edge_sp)

    dp = p["dec"][0]
    c1 = p["W_cell1"]
    dw = _mlp_weights(dp, ("W1", "W2", "W3")) + [
        _row(dp["ln1_g"]), _row(dp["ln1_b"]),
        dp["mix_in"]["W"], _row(dp["mix_in"]["b"]),
        dp["mix_out"]["W"], _row(dp["mix_out"]["b"]),
        _row(dp["ln2_g"]), _row(dp["ln2_b"]),
        p["W_recover"]["W"], _row(p["W_recover"]["b"]),
        _row(c1["W"][0:H, 0]), _row(c1["W"][H:2 * H, 0]),
        _row(c1["W"][2 * H:3 * H, 0]), c1["b"].reshape(1, 1),
        jnp.repeat(p["W_cell2"]["W"][:, 0], RB).reshape(ER, 1),
        p["W_cell2"]["b"].reshape(1, 1),
        p["W_cell3"]["W"], _row(p["W_cell3"]["b"])]

    gvd = p["W_recover"]["W"].shape[1]
    recover, logits = _mp_call(
        _dec_body, hv, he, nb3, dw,
        [jax.ShapeDtypeStruct((N, gvd), jnp.float32),
         jax.ShapeDtypeStruct((1, NSM1), jnp.float32)],
        [pl.BlockSpec((RB, gvd), lambda i: (i, 0)),
         pl.BlockSpec((1, NSM1), lambda i: (0, 0))])

    return recover[None], logits


# speedup vs baseline: 532.6465x; 1.7709x over previous
import functools

import jax
import jax.numpy as jnp
from jax import lax
from jax.experimental import pallas as pl
from jax.experimental.pallas import tpu as pltpu
from jax.experimental.pallas import tpu_sc as plsc

N = 2048
H = 128
KP = 8
KN = 8
KK = KP + KN
EDF = 64
GVDP = 256
NSM1 = 15
NK = N * KK
RB = 256
ER = KK * RB
NBLK = N // RB
NW = 32
CH = 128
SIGINV = EDF / 8.0
MUSTEP = 8.0 / (EDF - 1)
SQRT_HALF = 0.7071067811865476


def _gelu(x):
    return x * (0.5 * (1.0 + lax.erf(x * SQRT_HALF)))


def _ln(x, g, b):
    m = jnp.mean(x, axis=-1, keepdims=True)
    d = x - m
    v = jnp.mean(d * d, axis=-1, keepdims=True)
    return d * lax.rsqrt(v + 1e-5) * g + b


def _dot(a, b):
    return jnp.dot(a, b, preferred_element_type=jnp.float32)


def _feat_body(xb_ref, xt_ref, g_ref, wg_ref, wv_ref, bv_ref, we_ref, be_ref,
               eidx_ref, hv_ref, he_ref):
    xb = xb_ref[...]
    xt = xt_ref[...]
    x2a = jnp.sum(xt * xt, axis=0, keepdims=True)
    x2b = jnp.sum(xb * xb, axis=1, keepdims=True)
    d2 = x2b + x2a - 2.0 * _dot(xb, xt)
    dd = jnp.sqrt(jnp.maximum(d2, 0.0) + 1e-6)
    iota = lax.broadcasted_iota(jnp.int32, (RB, N), 1)
    mu = MUSTEP * lax.broadcasted_iota(jnp.int32, (1, EDF), 1).astype(jnp.float32)
    we = we_ref[...]
    be = be_ref[...]
    big = jnp.float32(3.0e38)

    def emit(k, val):
        e = jnp.exp(-jnp.square((val - mu) * SIGINV))
        he_ref[k, :, :] = _dot(e, we) + be

    cols = []
    work = dd
    for k in range(KP):
        mn = jnp.min(work, axis=1, keepdims=True)
        idx = jnp.min(jnp.where(work == mn, iota, jnp.int32(N)),
                      axis=1, keepdims=True)
        cols.append(idx)
        emit(k, mn)
        work = jnp.where(iota == idx, big, work)
    work = dd
    for k in range(KN):
        mx = jnp.max(work, axis=1, keepdims=True)
        idx = jnp.min(jnp.where(work == mx, iota, jnp.int32(N)),
                      axis=1, keepdims=True)
        cols.append(idx)
        emit(KP + k, mx)
        work = jnp.where(iota == idx, jnp.float32(-1.0), work)
    eidx_ref[...] = jnp.concatenate(cols, axis=1)

    v = _dot(g_ref[...], wg_ref[...])
    hv_ref[...] = _dot(v, wv_ref[...]) + bv_ref[...]


def _features(xp, xpt, genes_p, wg, wv, bv, we, be):
    full = lambda a: pl.BlockSpec(a.shape, lambda i: (0,) * a.ndim)
    return pl.pallas_call(
        _feat_body,
        grid=(NBLK,),
        in_specs=[
            pl.BlockSpec((RB, 8), lambda i: (i, 0)),
            full(xpt),
            pl.BlockSpec((RB, GVDP), lambda i: (i, 0)),
            full(wg), full(wv), full(bv), full(we), full(be),
        ],
        out_specs=[
            pl.BlockSpec((RB, KK), lambda i: (i, 0)),
            pl.BlockSpec((RB, H), lambda i: (i, 0)),
            pl.BlockSpec((KK, RB, H), lambda i: (0, i, 0)),
        ],
        out_shape=[
            jax.ShapeDtypeStruct((N, KK), jnp.int32),
            jax.ShapeDtypeStruct((N, H), jnp.float32),
            jax.ShapeDtypeStruct((KK, N, H), jnp.float32),
        ],
    )(xp, xpt, genes_p, wg, wv, bv, we, be)


def _sc_gather(table, idx2d):
    mesh = plsc.VectorSubcoreMesh(core_axis_name="c", subcore_axis_name="s")
    rows_per_w = NK // NW
    nch = rows_per_w // CH
    nb = 4

    @functools.partial(
        pl.kernel,
        mesh=mesh,
        out_type=jax.ShapeDtypeStruct((NK, H), jnp.float32),
        scratch_types=[
            pltpu.VMEM_SHARED((N, H), jnp.float32),
            pltpu.VMEM((nch, CH), jnp.int32),
        ] + [pltpu.VMEM((CH, H), jnp.float32) for _ in range(nb)] + [
            pltpu.SemaphoreType.DMA,
            pltpu.SemaphoreType.DMA,
        ],
    )
    def gather_k(table_hbm, idx_hbm, out_hbm, spm, idx_v, *rest):
        bufs = rest[:nb]
        gsem, wsem = rest[nb], rest[nb + 1]
        sid = lax.axis_index("s")
        wid = sid * 2 + lax.axis_index("c")
        base = wid * rows_per_w
        crow = wid * nch

        @pl.when(sid == 0)
        def _():
            pltpu.sync_copy(table_hbm, spm)

        pltpu.sync_copy(idx_hbm.at[pl.ds(crow, nch)], idx_v)
        plsc.subcore_barrier()
        gh = [None] * nch
        wh = [None] * nch
        for j in range(nch):
            if j >= nb:
                wh[j - nb].wait()
            gh[j] = pltpu.async_copy(spm.at[idx_v.at[j]], bufs[j % nb],
                                     gsem)
            if j >= 1:
                gh[j - 1].wait()
                wh[j - 1] = pltpu.async_copy(
                    bufs[(j - 1) % nb],
                    out_hbm.at[pl.ds(base + (j - 1) * CH, CH)], wsem)
        gh[nch - 1].wait()
        wh[nch - 1] = pltpu.async_copy(
            bufs[(nch - 1) % nb],
            out_hbm.at[pl.ds(base + (nch - 1) * CH, CH)], wsem)
        for j in range(max(0, nch - nb), nch):
            wh[j].wait()

    return gather_k(table, idx2d)


def _message_flat(hv, he, nbr, w1v, w1e, w1n, b1, w2, b2, w3, b3):
    a = _dot(hv, w1v) + b1
    a_rep = jnp.broadcast_to(a[None], (KK, RB, H)).reshape(ER, H)
    t = _dot(he, w1e) + _dot(nbr, w1n) + a_rep
    t = _gelu(t)
    t = _gelu(_dot(t, w2) + b2)
    return _dot(t, w3) + b3


def _node_common(hv, he_ref, nb_ref, w1v_ref, w1e_ref, w1n_ref, b1_ref,
                 w2_ref, b2_ref, w3_ref, b3_ref, g1_ref, gb1_ref,
                 mi_ref, mib_ref, mo_ref, mob_ref, g2_ref, gb2_ref):
    he = he_ref[...].reshape(ER, H)
    nbr = nb_ref[...].reshape(ER, H)
    t = _message_flat(hv, he, nbr, w1v_ref[...], w1e_ref[...], w1n_ref[...],
                      b1_ref[...], w2_ref[...], b2_ref[...], w3_ref[...],
                      b3_ref[...])
    acc = jnp.sum(t.reshape(KK, RB, H), axis=0)
    hv = _ln(hv + acc * (1.0 / KK), g1_ref[...], gb1_ref[...])
    ffn = _dot(_gelu(_dot(hv, mi_ref[...]) + mib_ref[...]), mo_ref[...]) \
        + mob_ref[...]
    return _ln(hv + ffn, g2_ref[...], gb2_ref[...])


def _node_body(hv_ref, he_ref, nb_ref, *refs):
    out_ref = refs[-1]
    out_ref[...] = _node_common(hv_ref[...], he_ref, nb_ref, *refs[:-1])


def _edge_body(hv_ref, he_ref, nb_ref, w1v_ref, w1e_ref, w1n_ref, b1_ref,
               w2_ref, b2_ref, w3_ref, b3_ref, g3_ref, gb3_ref, out_ref):
    he = he_ref[...].reshape(ER, H)
    nbr = nb_ref[...].reshape(ER, H)
    t = _message_flat(hv_ref[...], he, nbr, w1v_ref[...], w1e_ref[...],
                      w1n_ref[...], b1_ref[...], w2_ref[...], b2_ref[...],
                      w3_ref[...], b3_ref[...])
    out_ref[...] = _ln(he + t, g3_ref[...], gb3_ref[...]).reshape(KK, RB, H)


def _dec_body(hv_ref, he_ref, nb_ref, w1v_ref, w1e_ref, w1n_ref, b1_ref,
              w2_ref, b2_ref, w3_ref, b3_ref, g1_ref, gb1_ref,
              mi_ref, mib_ref, mo_ref, mob_ref, g2_ref, gb2_ref,
              wr_ref, br_ref,
              rv1_ref, re1_ref, rn1_ref, rb1_ref, w2rep_ref, rb2_ref,
              w3c_ref, rb3_ref,
              rec_ref, log_ref):
    hv = hv_ref[...]
    he = he_ref[...].reshape(ER, H)
    nbr = nb_ref[...].reshape(ER, H)

    av = jnp.sum(hv * rv1_ref[...], axis=1, keepdims=True)
    s = jnp.sum(he * re1_ref[...], axis=1, keepdims=True) \
        + jnp.sum(nbr * rn1_ref[...], axis=1, keepdims=True)
    av_rep = jnp.broadcast_to(av[None], (KK, RB, 1)).reshape(ER, 1)
    h = _gelu(s + av_rep + rb1_ref[...])
    tsum = jnp.sum((h * w2rep_ref[...]).reshape(KK, RB, 1), axis=0)
    t2 = _gelu(tsum + rb2_ref[...])
    i = pl.program_id(0)
    w3c = w3c_ref[pl.ds(i * RB, RB), :]
    part = jnp.sum(t2 * w3c, axis=0, keepdims=True)

    @pl.when(i == 0)
    def _():
        log_ref[...] = part + rb3_ref[...]

    @pl.when(i != 0)
    def _():
        log_ref[...] += part

    hv2 = _node_common(hv, he_ref, nb_ref, w1v_ref, w1e_ref, w1n_ref, b1_ref,
                       w2_ref, b2_ref, w3_ref, b3_ref, g1_ref, gb1_ref,
                       mi_ref, mib_ref, mo_ref, mob_ref, g2_ref, gb2_ref)
    rec_ref[...] = _dot(hv2, wr_ref[...]) + br_ref[...]


def _mp_call(body, hv, he, nb3, weights, out_shape, out_spec,
             scratch_shapes=()):
    full = lambda a: pl.BlockSpec(a.shape, lambda i: (0,) * a.ndim)
    return pl.pallas_call(
        body,
        grid=(NBLK,),
        in_specs=[
            pl.BlockSpec((RB, H), lambda i: (i, 0)),
            pl.BlockSpec((KK, RB, H), lambda i: (0, i, 0)),
            pl.BlockSpec((KK, RB, H), lambda i: (0, i, 0)),
        ] + [full(w) for w in weights],
        out_specs=out_spec,
        out_shape=out_shape,
        scratch_shapes=list(scratch_shapes),
    )(hv, he, nb3, *weights)


def _row(v):
    return v.reshape(1, -1)


def _mlp_weights(p, names):
    w1, w2, w3 = (p[n] for n in names)
    return [w1["W"][0:H], w1["W"][H:2 * H], w1["W"][2 * H:3 * H],
            _row(w1["b"]), w2["W"], _row(w2["b"]), w3["W"], _row(w3["b"])]


def kernel(genes, X, mask, params):
    p = params
    g2 = jnp.pad(genes[0], ((0, 0), (0, GVDP - genes.shape[-1])))
    xp = jnp.pad(X[0], ((0, 0), (0, 8 - X.shape[-1])))
    xpt = xp.T
    wg = jnp.pad(p["W_genes"], ((0, GVDP - p["W_genes"].shape[0]), (0, 0)))

    eidx, hv, he = _features(
        xp, xpt, g2, wg, p["W_v"]["W"], _row(p["W_v"]["b"]),
        p["W_e"]["W"], _row(p["W_e"]["b"]))
    idx2d = eidx.T.reshape(NK // CH, CH)

    node_sp = pl.BlockSpec((RB, H), lambda i: (i, 0))
    edge_sp = pl.BlockSpec((KK, RB, H), lambda i: (0, i, 0))
    node_sh = jax.ShapeDtypeStruct((N, H), jnp.float32)
    edge_sh = jax.ShapeDtypeStruct((KK, N, H), jnp.float32)

    nb3 = _sc_gather(hv, idx2d).reshape(KK, N, H)
    for ep in p["enc"]:
        nw = _mlp_weights(ep, ("W1", "W2", "W3")) + [
            _row(ep["ln1_g"]), _row(ep["ln1_b"]),
            ep["mix_in"]["W"], _row(ep["mix_in"]["b"]),
            ep["mix_out"]["W"], _row(ep["mix_out"]["b"]),
            _row(ep["ln2_g"]), _row(ep["ln2_b"])]
        hv = _mp_call(_node_body, hv, he, nb3, nw, node_sh, node_sp)
        ew = _mlp_weights(ep, ("W11", "W12", "W13")) + [
            _row(ep["ln3_g"]), _row(ep["ln3_b"])]
        nb3 = _sc_gather(hv, idx2d).reshape(KK, N, H)
        he = _mp_call(_edge_body, hv, he, nb3, ew, edge_sh, edge_sp)

    dp = p["dec"][0]
    c1 = p["W_cell1"]
    dw = _mlp_weights(dp, ("W1", "W2", "W3")) + [
        _row(dp["ln1_g"]), _row(dp["ln1_b"]),
        dp["mix_in"]["W"], _row(dp["mix_in"]["b"]),
        dp["mix_out"]["W"], _row(dp["mix_out"]["b"]),
        _row(dp["ln2_g"]), _row(dp["ln2_b"]),
        p["W_recover"]["W"], _row(p["W_recover"]["b"]),
        _row(c1["W"][0:H, 0]), _row(c1["W"][H:2 * H, 0]),
        _row(c1["W"][2 * H:3 * H, 0]), c1["b"].reshape(1, 1),
        jnp.repeat(p["W_cell2"]["W"][:, 0], RB).reshape(ER, 1),
        p["W_cell2"]["b"].reshape(1, 1),
        p["W_cell3"]["W"], _row(p["W_cell3"]["b"])]

    gvd = p["W_recover"]["W"].shape[1]
    recover, logits = _mp_call(
        _dec_body, hv, he, nb3, dw,
        [jax.ShapeDtypeStruct((N, gvd), jnp.float32),
         jax.ShapeDtypeStruct((1, NSM1), jnp.float32)],
        [pl.BlockSpec((RB, gvd), lambda i: (i, 0)),
         pl.BlockSpec((1, NSM1), lambda i: (0, 0))])

    return recover[None], logits
